# Initial kernel scaffold; baseline (speedup 1.0000x reference)
import jax
import jax.numpy as jnp
from jax.experimental import pallas as pl


def kernel(entity_embedding1, entity_embedding2, relation_embedding1, relation_embedding2, heads, relations, tails):
    raise NotImplementedError("write your pallas kernel here")


# speedup vs baseline: 1.9502x; 1.9502x over previous
import functools

import numpy as np

import jax
import jax.numpy as jnp
from jax import lax
from jax.experimental import pallas as pl
from jax.experimental.pallas import tpu as pltpu
from jax.experimental.pallas import tpu_sc as plsc

D = 128
B = 16384
NC = 2
NS = 16
NW = NC * NS
BPW = B // NW
CH = 64
NCHUNK = BPW // CH
NSL = D // 16

_GATHER_DNUMS = lax.GatherDimensionNumbers(
    offset_dims=(), collapsed_slice_dims=(0,), start_index_map=(0,))


def _lane_permute(v, idx):
    return lax.gather(v, idx[:, None], _GATHER_DNUMS, slice_sizes=(1,),
                      mode=lax.GatherScatterMode.PROMISE_IN_BOUNDS)


@functools.partial(
    pl.kernel,
    out_type=jax.ShapeDtypeStruct((B,), jnp.float32),
    mesh=plsc.VectorSubcoreMesh(core_axis_name="c", subcore_axis_name="s"),
    scratch_types=[
        pltpu.VMEM((BPW,), jnp.int32),
        pltpu.VMEM((BPW,), jnp.int32),
        pltpu.VMEM((BPW,), jnp.int32),
        pltpu.VMEM((CH, D), jnp.float32),
        pltpu.VMEM((CH, D), jnp.float32),
        pltpu.VMEM((CH, D), jnp.float32),
        pltpu.VMEM((CH, D), jnp.float32),
        pltpu.VMEM((CH, D), jnp.float32),
        pltpu.VMEM((CH, D), jnp.float32),
        pltpu.VMEM((BPW,), jnp.float32),
        pltpu.SemaphoreType.DMA,
    ],
)
def _sc_complex_score(e1_hbm, e2_hbm, r1_hbm, r2_hbm,
                      heads_hbm, rels_hbm, tails_hbm, out_hbm,
                      h_v, r_v, t_v, e11, e12, e21, e22, rr1, rr2,
                      sc_v, sem):
    wid = lax.axis_index("s") * NC + lax.axis_index("c")
    base = wid * BPW
    pltpu.sync_copy(heads_hbm.at[pl.ds(base, BPW)], h_v)
    pltpu.sync_copy(rels_hbm.at[pl.ds(base, BPW)], r_v)
    pltpu.sync_copy(tails_hbm.at[pl.ds(base, BPW)], t_v)

    def chunk_body(c, carry):
        off = c * CH
        hs = h_v.at[pl.ds(off, CH)]
        rs = r_v.at[pl.ds(off, CH)]
        ts = t_v.at[pl.ds(off, CH)]
        cps = [
            pltpu.async_copy(e1_hbm.at[hs], e11, sem),
            pltpu.async_copy(e2_hbm.at[hs], e12, sem),
            pltpu.async_copy(e1_hbm.at[ts], e21, sem),
            pltpu.async_copy(e2_hbm.at[ts], e22, sem),
            pltpu.async_copy(r1_hbm.at[rs], rr1, sem),
            pltpu.async_copy(r2_hbm.at[rs], rr2, sem),
        ]
        for cp in cps:
            cp.wait()

        def group_body(g, rc):
            lane = lax.iota(jnp.int32, 16)
            perms = [lane ^ k for k in (1, 2, 4, 8)]
            w = jnp.zeros((16,), jnp.float32)
            for j in range(16):
                i = g * 16 + j
                accp = jnp.zeros((16,), jnp.float32)
                accq = jnp.zeros((16,), jnp.float32)
                for s in range(NSL):
                    sl = pl.ds(s * 16, 16)
                    a = e11[i, sl]
                    b = e12[i, sl]
                    u = e21[i, sl]
                    v = e22[i, sl]
                    p1 = rr1[i, sl]
                    p2 = rr2[i, sl]
                    accp = accp + p1 * (a * u + b * v)
                    accq = accq + p2 * (a * v - b * u)
                t = accp + accq
                for p in perms:
                    t = t + _lane_permute(t, p)
                w = jnp.where(lane == j, t, w)
            sc_v[pl.ds(off + g * 16, 16)] = 1.0 / (1.0 + jnp.exp(-w))
            return rc

        return lax.fori_loop(0, CH // 16, group_body, carry)

    lax.fori_loop(0, NCHUNK, chunk_body, 0)
    pltpu.sync_copy(sc_v, out_hbm.at[pl.ds(base, BPW)])


def kernel(entity_embedding1, entity_embedding2, relation_embedding1,
           relation_embedding2, heads, relations, tails):
    return _sc_complex_score(
        entity_embedding1, entity_embedding2,
        relation_embedding1, relation_embedding2,
        heads.astype(jnp.int32), relations.astype(jnp.int32),
        tails.astype(jnp.int32))
